# baseline (speedup 1.0000x reference)
import functools

import jax
import jax.numpy as jnp
from jax import lax
from jax.experimental import pallas as pl
from jax.experimental.pallas import tpu as pltpu
from jax.experimental.pallas import tpu_sc as plsc

NN = 10000
EE = 320000
GG = 64
NC, NS = 2, 16
NP = 10240
STRIPE = NP // NS
KC = 128
E2 = 327680
NROW = E2 // KC
BN = 400
NBLK = NN // BN

_f32 = jnp.float32


@functools.cache
def _make_deg():
    mesh = plsc.VectorSubcoreMesh(core_axis_name="c", subcore_axis_name="s",
                                  num_cores=NC, num_subcores=NS)
    return pl.kernel(
        _deg_body,
        out_type=jax.ShapeDtypeStruct((NC * NP, 128), _f32),
        mesh=mesh,
        scratch_types=[
            pltpu.VMEM((NROW // (NC * NS), KC), jnp.int32),
            pltpu.VMEM((KC, 128), _f32),
            pltpu.VMEM_SHARED((NP, 128), _f32),
        ],
    )


def _deg_body(dst2, ones_hbm, zeros_hbm, parts, dstv, onesv, deg_sp):
    c = lax.axis_index("c")
    s = lax.axis_index("s")
    rpt = NROW // (NC * NS)
    pltpu.sync_copy(dst2.at[pl.ds((c * NS + s) * rpt, rpt)], dstv)
    pltpu.sync_copy(ones_hbm, onesv)
    pltpu.sync_copy(zeros_hbm.at[pl.ds(s * STRIPE, STRIPE)],
                    deg_sp.at[pl.ds(s * STRIPE, STRIPE)])
    plsc.subcore_barrier()

    def body(j, carry):
        pltpu.sync_copy(onesv, deg_sp.at[dstv.at[j]], add=True)
        return carry

    lax.fori_loop(0, rpt, body, 0)
    plsc.subcore_barrier()
    pltpu.sync_copy(deg_sp.at[pl.ds(s * STRIPE, STRIPE)],
                    parts.at[pl.ds(c * NP + s * STRIPE, STRIPE)])


@functools.cache
def _make_agg():
    mesh = plsc.VectorSubcoreMesh(core_axis_name="c", subcore_axis_name="s",
                                  num_cores=NC, num_subcores=NS)
    return pl.kernel(
        _agg_body,
        out_type=jax.ShapeDtypeStruct((NC * NP, 128), _f32),
        mesh=mesh,
        scratch_types=[
            pltpu.VMEM((KC,), jnp.int32),
            pltpu.VMEM((KC,), jnp.int32),
            pltpu.VMEM((KC,), jnp.int32),
            pltpu.VMEM((KC,), jnp.int32),
            pltpu.VMEM((KC, 128), _f32),
            pltpu.VMEM((KC, 128), _f32),
            pltpu.VMEM_SHARED((NP, 128), _f32),
            pltpu.SemaphoreType.DMA,
            pltpu.SemaphoreType.DMA,
            pltpu.SemaphoreType.DMA,
            pltpu.SemaphoreType.DMA,
        ],
    )


def _agg_body(g, src2, dst2, zeros_hbm, acc, src0, src1, dst0, dst1, rows0,
              rows1, acc_sp, semi0, semi1, semg0, semg1):
    c = lax.axis_index("c")
    s = lax.axis_index("s")
    rpt = NROW // NS
    srow = c * NROW + s * rpt
    drow = s * rpt
    pltpu.sync_copy(zeros_hbm.at[pl.ds(s * STRIPE, STRIPE)],
                    acc_sp.at[pl.ds(s * STRIPE, STRIPE)])
    plsc.subcore_barrier()
    srcb = (src0, src1)
    dstb = (dst0, dst1)
    rows = (rows0, rows1)
    semi = (semi0, semi1)
    semg = (semg0, semg1)

    def idx_load(j, b):
        pltpu.async_copy(src2.at[srow + j], srcb[b], semi[b])
        pltpu.async_copy(dst2.at[drow + j], dstb[b], semi[b])

    def idx_wait(b):
        pltpu.make_async_copy(src2.at[0], srcb[b], semi[b]).wait()
        pltpu.make_async_copy(dst2.at[0], dstb[b], semi[b]).wait()

    idx_load(0, 0)
    idx_wait(0)
    pltpu.async_copy(g.at[src0], rows0, semg0)
    idx_load(1, 1)

    def body(t, carry):
        for b in range(2):
            j = 2 * t + b
            nb = 1 - b

            @pl.when(j + 1 < rpt)
            def _():
                idx_wait(nb)
                pltpu.async_copy(g.at[srcb[nb]], rows[nb], semg[nb])

            pltpu.make_async_copy(g.at[srcb[b]], rows[b], semg[b]).wait()
            pltpu.sync_copy(rows[b], acc_sp.at[dstb[b]], add=True)

            @pl.when(j + 2 < rpt)
            def _():
                idx_load(j + 2, b)

        return carry

    lax.fori_loop(0, rpt // 2, body, 0)
    plsc.subcore_barrier()
    pltpu.sync_copy(acc_sp.at[pl.ds(s * STRIPE, STRIPE)],
                    acc.at[pl.ds(c * NP + s * STRIPE, STRIPE)])


def _mm1_body(parts_ref, x_ref, w_ref, g_ref, dis_ref):
    deg = parts_ref[0, :, 0:1] + parts_ref[1, :, 0:1] + 1.0
    dis = lax.rsqrt(deg)
    dis_ref[...] = dis
    t = jnp.dot(x_ref[...], w_ref[...], preferred_element_type=_f32)
    gt = t * dis
    g_ref[0] = gt[:, :128]
    g_ref[1] = gt[:, 128:]


def _mm1(parts, x, w1):
    return pl.pallas_call(
        _mm1_body,
        grid=(NBLK,),
        in_specs=[
            pl.BlockSpec((NC, BN, 128), lambda i: (0, i, 0)),
            pl.BlockSpec((BN, 128), lambda i: (i, 0)),
            pl.BlockSpec((128, 256), lambda i: (0, 0)),
        ],
        out_specs=[
            pl.BlockSpec((NC, BN, 128), lambda i: (0, i, 0)),
            pl.BlockSpec((BN, 1), lambda i: (i, 0)),
        ],
        out_shape=[
            jax.ShapeDtypeStruct((NC, NP, 128), _f32),
            jax.ShapeDtypeStruct((NN, 1), _f32),
        ],
    )(parts, x, w1)


def _combine_body(acc_ref, g_ref, dis_ref, b_ref, w_ref, gout_ref):
    h = jnp.concatenate([acc_ref[0] + g_ref[0], acc_ref[1] + g_ref[1]], axis=1)
    h = jnp.maximum(h * dis_ref[...] + b_ref[...], 0.0)
    t = jnp.dot(h, w_ref[...], preferred_element_type=_f32)
    gt = t * dis_ref[...]
    gout_ref[0] = gt[:, :128]
    gout_ref[1] = gt[:, 128:]


def _combine(acc, g, dis, b, w):
    return pl.pallas_call(
        _combine_body,
        grid=(NBLK,),
        in_specs=[
            pl.BlockSpec((NC, BN, 128), lambda i: (0, i, 0)),
            pl.BlockSpec((NC, BN, 128), lambda i: (0, i, 0)),
            pl.BlockSpec((BN, 1), lambda i: (i, 0)),
            pl.BlockSpec((1, 256), lambda i: (0, 0)),
            pl.BlockSpec((256, 256), lambda i: (0, 0)),
        ],
        out_specs=pl.BlockSpec((NC, BN, 128), lambda i: (0, i, 0)),
        out_shape=jax.ShapeDtypeStruct((NC, NP, 128), _f32),
    )(acc, g, dis, b, w)


def _pool_body(acc_ref, g_ref, dis_ref, b_ref, batch_ref, wl_ref, bl_ref,
               emb_ref, logits_ref, sums_ref, counts_ref):
    i = pl.program_id(0)
    e = jnp.concatenate([acc_ref[0] + g_ref[0], acc_ref[1] + g_ref[1]], axis=1)
    e = e * dis_ref[...] + b_ref[...]
    emb_ref[...] = e
    m = (batch_ref[...] ==
         lax.broadcasted_iota(jnp.int32, (1, GG), 1)).astype(_f32)
    part_sums = lax.dot_general(m, e, (((0,), (0,)), ((), ())),
                                preferred_element_type=_f32)
    part_counts = lax.dot_general(m, jnp.ones((BN, 128), _f32),
                                  (((0,), (0,)), ((), ())),
                                  preferred_element_type=_f32)

    @pl.when(i == 0)
    def _():
        sums_ref[...] = jnp.zeros_like(sums_ref)
        counts_ref[...] = jnp.zeros_like(counts_ref)

    sums_ref[...] += part_sums
    counts_ref[...] += part_counts

    @pl.when(i == NBLK - 1)
    def _():
        pooled = sums_ref[...] / jnp.maximum(counts_ref[...][:, 0:1], 1.0)
        logits_ref[...] = (jnp.dot(pooled, wl_ref[...],
                                   preferred_element_type=_f32) + bl_ref[...])


def _pool(acc, g, dis, b3, batch2d, wl, bl):
    return pl.pallas_call(
        _pool_body,
        grid=(NBLK,),
        in_specs=[
            pl.BlockSpec((NC, BN, 128), lambda i: (0, i, 0)),
            pl.BlockSpec((NC, BN, 128), lambda i: (0, i, 0)),
            pl.BlockSpec((BN, 1), lambda i: (i, 0)),
            pl.BlockSpec((1, 256), lambda i: (0, 0)),
            pl.BlockSpec((BN, 1), lambda i: (i, 0)),
            pl.BlockSpec((256, 2), lambda i: (0, 0)),
            pl.BlockSpec((1, 2), lambda i: (0, 0)),
        ],
        out_specs=[
            pl.BlockSpec((BN, 256), lambda i: (i, 0)),
            pl.BlockSpec((GG, 2), lambda i: (0, 0)),
        ],
        out_shape=[
            jax.ShapeDtypeStruct((NN, 256), _f32),
            jax.ShapeDtypeStruct((GG, 2), _f32),
        ],
        scratch_shapes=[
            pltpu.VMEM((GG, 256), _f32),
            pltpu.VMEM((GG, 128), _f32),
        ],
    )(acc, g, dis, b3, batch2d, wl, bl)


def kernel(x, edge_index, batch, W1, b1, W2, b2, W3, b3, Wl, bl):
    zeros128 = jnp.zeros((NP, 128), _f32)
    ones128 = jnp.ones((KC, 128), _f32)

    pad = E2 - EE
    order = jnp.argsort(edge_index[0])
    src_pad = jnp.concatenate(
        [edge_index[0][order], jnp.zeros((pad,), jnp.int32)])
    dst_pad = jnp.concatenate(
        [edge_index[1][order], jnp.full((pad,), NP - 1, jnp.int32)])
    src2 = jnp.concatenate([src_pad, src_pad + NP]).reshape(2 * NROW, KC)
    dst2 = dst_pad.reshape(NROW, KC)
    deg_k = _make_deg()
    agg_k = _make_agg()
    parts = deg_k(dst2, ones128, zeros128).reshape(NC, NP, 128)
    g1, dis = _mm1(parts, x, W1)
    a1 = agg_k(g1.reshape(NC * NP, 128), src2, dst2, zeros128)
    g2 = _combine(a1.reshape(NC, NP, 128), g1, dis, b1.reshape(1, -1), W2)
    a2 = agg_k(g2.reshape(NC * NP, 128), src2, dst2, zeros128)
    g3 = _combine(a2.reshape(NC, NP, 128), g2, dis, b2.reshape(1, -1), W3)
    a3 = agg_k(g3.reshape(NC * NP, 128), src2, dst2, zeros128)
    emb, logits = _pool(a3.reshape(NC, NP, 128), g3, dis, b3.reshape(1, -1),
                        batch.reshape(-1, 1), Wl, bl.reshape(1, -1))
    return (logits, emb)


# speedup vs baseline: 1.4161x; 1.4161x over previous
import functools

import jax
import jax.numpy as jnp
from jax import lax
from jax.experimental import pallas as pl
from jax.experimental.pallas import tpu as pltpu
from jax.experimental.pallas import tpu_sc as plsc

NN = 10000
EE = 320000
GG = 64
NC, NS = 2, 16
NP = 10240
STRIPE = NP // NS
KC = 128
E2 = 327680
NROW = E2 // KC
BN = 400
NBLK = NN // BN

_f32 = jnp.float32


@functools.cache
def _make_deg():
    mesh = plsc.VectorSubcoreMesh(core_axis_name="c", subcore_axis_name="s",
                                  num_cores=NC, num_subcores=NS)
    return pl.kernel(
        _deg_body,
        out_type=jax.ShapeDtypeStruct((NC * NP, 128), _f32),
        mesh=mesh,
        scratch_types=[
            pltpu.VMEM((NROW // (NC * NS), KC), jnp.int32),
            pltpu.VMEM((KC, 128), _f32),
            pltpu.VMEM_SHARED((NP, 128), _f32),
        ],
    )


def _deg_body(dst2, ones_hbm, zeros_hbm, parts, dstv, onesv, deg_sp):
    c = lax.axis_index("c")
    s = lax.axis_index("s")
    rpt = NROW // (NC * NS)
    pltpu.sync_copy(dst2.at[pl.ds((c * NS + s) * rpt, rpt)], dstv)
    pltpu.sync_copy(ones_hbm, onesv)
    pltpu.sync_copy(zeros_hbm.at[pl.ds(s * STRIPE, STRIPE)],
                    deg_sp.at[pl.ds(s * STRIPE, STRIPE)])
    plsc.subcore_barrier()

    def body(j, carry):
        pltpu.sync_copy(onesv, deg_sp.at[dstv.at[j]], add=True)
        return carry

    lax.fori_loop(0, rpt, body, 0)
    plsc.subcore_barrier()
    pltpu.sync_copy(deg_sp.at[pl.ds(s * STRIPE, STRIPE)],
                    parts.at[pl.ds(c * NP + s * STRIPE, STRIPE)])


@functools.cache
def _make_agg():
    mesh = plsc.VectorSubcoreMesh(core_axis_name="c", subcore_axis_name="s",
                                  num_cores=NC, num_subcores=NS)
    return pl.kernel(
        _agg_body,
        out_type=jax.ShapeDtypeStruct((NC * NP, 128), _f32),
        mesh=mesh,
        scratch_types=[
            pltpu.VMEM((KC,), jnp.int32),
            pltpu.VMEM((KC,), jnp.int32),
            pltpu.VMEM((KC,), jnp.int32),
            pltpu.VMEM((KC,), jnp.int32),
            pltpu.VMEM((KC, 128), _f32),
            pltpu.VMEM((KC, 128), _f32),
            pltpu.VMEM_SHARED((NP, 128), _f32),
            pltpu.SemaphoreType.DMA,
            pltpu.SemaphoreType.DMA,
            pltpu.SemaphoreType.DMA,
            pltpu.SemaphoreType.DMA,
        ],
    )


def _agg_body(g, src2, dst2, zeros_hbm, acc, src0, src1, dst0, dst1, rows0,
              rows1, acc_sp, semi0, semi1, semg0, semg1):
    c = lax.axis_index("c")
    s = lax.axis_index("s")
    rpt = NROW // NS
    srow = c * NROW + s * rpt
    drow = s * rpt
    pltpu.sync_copy(zeros_hbm.at[pl.ds(s * STRIPE, STRIPE)],
                    acc_sp.at[pl.ds(s * STRIPE, STRIPE)])
    plsc.subcore_barrier()
    srcb = (src0, src1)
    dstb = (dst0, dst1)
    rows = (rows0, rows1)
    semi = (semi0, semi1)
    semg = (semg0, semg1)

    def idx_load(j, b):
        pltpu.async_copy(src2.at[srow + j], srcb[b], semi[b])
        pltpu.async_copy(dst2.at[drow + j], dstb[b], semi[b])

    def idx_wait(b):
        pltpu.make_async_copy(src2.at[0], srcb[b], semi[b]).wait()
        pltpu.make_async_copy(dst2.at[0], dstb[b], semi[b]).wait()

    idx_load(0, 0)
    idx_wait(0)
    pltpu.async_copy(g.at[src0], rows0, semg0)
    idx_load(1, 1)

    def body(t, carry):
        for b in range(2):
            j = 2 * t + b
            nb = 1 - b

            @pl.when(j + 1 < rpt)
            def _():
                idx_wait(nb)
                pltpu.async_copy(g.at[srcb[nb]], rows[nb], semg[nb])

            pltpu.make_async_copy(g.at[srcb[b]], rows[b], semg[b]).wait()
            pltpu.sync_copy(rows[b], acc_sp.at[dstb[b]], add=True)

            @pl.when(j + 2 < rpt)
            def _():
                idx_load(j + 2, b)

        return carry

    lax.fori_loop(0, rpt // 2, body, 0)
    plsc.subcore_barrier()
    pltpu.sync_copy(acc_sp.at[pl.ds(s * STRIPE, STRIPE)],
                    acc.at[pl.ds(c * NP + s * STRIPE, STRIPE)])


def _mm1_body(parts_ref, x_ref, w_ref, g_ref, dis_ref):
    deg = parts_ref[0, :, 0:1] + parts_ref[1, :, 0:1] + 1.0
    dis = lax.rsqrt(deg)
    dis_ref[...] = dis
    t = jnp.dot(x_ref[...], w_ref[...], preferred_element_type=_f32)
    gt = t * dis
    g_ref[0] = gt[:, :128]
    g_ref[1] = gt[:, 128:]


def _mm1(parts, x, w1):
    return pl.pallas_call(
        _mm1_body,
        grid=(NBLK,),
        in_specs=[
            pl.BlockSpec((NC, BN, 128), lambda i: (0, i, 0)),
            pl.BlockSpec((BN, 128), lambda i: (i, 0)),
            pl.BlockSpec((128, 256), lambda i: (0, 0)),
        ],
        out_specs=[
            pl.BlockSpec((NC, BN, 128), lambda i: (0, i, 0)),
            pl.BlockSpec((BN, 1), lambda i: (i, 0)),
        ],
        out_shape=[
            jax.ShapeDtypeStruct((NC, NP, 128), _f32),
            jax.ShapeDtypeStruct((NN, 1), _f32),
        ],
    )(parts, x, w1)


def _combine_body(acc_ref, g_ref, dis_ref, b_ref, w_ref, gout_ref):
    h = jnp.concatenate([acc_ref[0] + g_ref[0], acc_ref[1] + g_ref[1]], axis=1)
    h = jnp.maximum(h * dis_ref[...] + b_ref[...], 0.0)
    t = jnp.dot(h, w_ref[...], preferred_element_type=_f32)
    gt = t * dis_ref[...]
    gout_ref[0] = gt[:, :128]
    gout_ref[1] = gt[:, 128:]


def _combine(acc, g, dis, b, w):
    return pl.pallas_call(
        _combine_body,
        grid=(NBLK,),
        in_specs=[
            pl.BlockSpec((NC, BN, 128), lambda i: (0, i, 0)),
            pl.BlockSpec((NC, BN, 128), lambda i: (0, i, 0)),
            pl.BlockSpec((BN, 1), lambda i: (i, 0)),
            pl.BlockSpec((1, 256), lambda i: (0, 0)),
            pl.BlockSpec((256, 256), lambda i: (0, 0)),
        ],
        out_specs=pl.BlockSpec((NC, BN, 128), lambda i: (0, i, 0)),
        out_shape=jax.ShapeDtypeStruct((NC, NP, 128), _f32),
    )(acc, g, dis, b, w)


def _pool_body(acc_ref, g_ref, dis_ref, b_ref, batch_ref, wl_ref, bl_ref,
               emb_ref, logits_ref, sums_ref, counts_ref):
    i = pl.program_id(0)
    e = jnp.concatenate([acc_ref[0] + g_ref[0], acc_ref[1] + g_ref[1]], axis=1)
    e = e * dis_ref[...] + b_ref[...]
    emb_ref[...] = e
    m = (batch_ref[...] ==
         lax.broadcasted_iota(jnp.int32, (1, GG), 1)).astype(_f32)
    part_sums = lax.dot_general(m, e, (((0,), (0,)), ((), ())),
                                preferred_element_type=_f32)
    part_counts = lax.dot_general(m, jnp.ones((BN, 128), _f32),
                                  (((0,), (0,)), ((), ())),
                                  preferred_element_type=_f32)

    @pl.when(i == 0)
    def _():
        sums_ref[...] = jnp.zeros_like(sums_ref)
        counts_ref[...] = jnp.zeros_like(counts_ref)

    sums_ref[...] += part_sums
    counts_ref[...] += part_counts

    @pl.when(i == NBLK - 1)
    def _():
        pooled = sums_ref[...] / jnp.maximum(counts_ref[...][:, 0:1], 1.0)
        logits_ref[...] = (jnp.dot(pooled, wl_ref[...],
                                   preferred_element_type=_f32) + bl_ref[...])


def _pool(acc, g, dis, b3, batch2d, wl, bl):
    return pl.pallas_call(
        _pool_body,
        grid=(NBLK,),
        in_specs=[
            pl.BlockSpec((NC, BN, 128), lambda i: (0, i, 0)),
            pl.BlockSpec((NC, BN, 128), lambda i: (0, i, 0)),
            pl.BlockSpec((BN, 1), lambda i: (i, 0)),
            pl.BlockSpec((1, 256), lambda i: (0, 0)),
            pl.BlockSpec((BN, 1), lambda i: (i, 0)),
            pl.BlockSpec((256, 2), lambda i: (0, 0)),
            pl.BlockSpec((1, 2), lambda i: (0, 0)),
        ],
        out_specs=[
            pl.BlockSpec((BN, 256), lambda i: (i, 0)),
            pl.BlockSpec((GG, 2), lambda i: (0, 0)),
        ],
        out_shape=[
            jax.ShapeDtypeStruct((NN, 256), _f32),
            jax.ShapeDtypeStruct((GG, 2), _f32),
        ],
        scratch_shapes=[
            pltpu.VMEM((GG, 256), _f32),
            pltpu.VMEM((GG, 128), _f32),
        ],
    )(acc, g, dis, b3, batch2d, wl, bl)


def kernel(x, edge_index, batch, W1, b1, W2, b2, W3, b3, Wl, bl):
    zeros128 = jnp.zeros((NP, 128), _f32)
    ones128 = jnp.ones((KC, 128), _f32)

    pad = E2 - EE
    src_pad = jnp.concatenate([edge_index[0], jnp.zeros((pad,), jnp.int32)])
    dst_pad = jnp.concatenate(
        [edge_index[1], jnp.full((pad,), NP - 1, jnp.int32)])
    src2 = jnp.concatenate([src_pad, src_pad + NP]).reshape(2 * NROW, KC)
    dst2 = dst_pad.reshape(NROW, KC)
    deg_k = _make_deg()
    agg_k = _make_agg()
    parts = deg_k(dst2, ones128, zeros128).reshape(NC, NP, 128)
    g1, dis = _mm1(parts, x, W1)
    a1 = agg_k(g1.reshape(NC * NP, 128), src2, dst2, zeros128)
    g2 = _combine(a1.reshape(NC, NP, 128), g1, dis, b1.reshape(1, -1), W2)
    a2 = agg_k(g2.reshape(NC * NP, 128), src2, dst2, zeros128)
    g3 = _combine(a2.reshape(NC, NP, 128), g2, dis, b2.reshape(1, -1), W3)
    a3 = agg_k(g3.reshape(NC * NP, 128), src2, dst2, zeros128)
    emb, logits = _pool(a3.reshape(NC, NP, 128), g3, dis, b3.reshape(1, -1),
                        batch.reshape(-1, 1), Wl, bl.reshape(1, -1))
    return (logits, emb)
